# baseline (speedup 1.0000x reference)
import jax
import jax.numpy as jnp
from jax.experimental import pallas as pl
from jax.experimental.pallas import tpu as pltpu

_SIZE = 26
_D = 2048
_LEVELS = 100
_LP = 104
_K = _SIZE * _LP
_BATCH = 1024
_BB = 256


def _body(x_ref, pos_ref, lev_ref, out_ref, w_ref, e_ref, lmod_ref):
    i = pl.program_id(0)

    @pl.when(i == 0)
    def _build_tables():
        lev = lev_ref[...]
        for s in range(_SIZE):
            p = pos_ref[s:s + 1, :]
            w_ref[s * _LP:(s + 1) * _LP, :] = (lev * p).astype(jnp.int8)
        js = jax.lax.broadcasted_iota(jnp.int32, (32, _K), 1) // _LP
        ss = jax.lax.broadcasted_iota(jnp.int32, (32, _K), 0)
        e_ref[...] = jnp.where(js == ss, 1.0, 0.0).astype(jnp.bfloat16)
        cols = jax.lax.broadcasted_iota(jnp.int32, (_BB, _K), 1)
        lmod_ref[...] = (cols % _LP).astype(jnp.float32)

    xb = x_ref[...]
    idx = jnp.clip(jnp.round(xb * (_LEVELS - 1)).astype(jnp.int32),
                   0, _LEVELS - 1)
    idx32 = jnp.pad(idx.astype(jnp.bfloat16), ((0, 0), (0, 32 - _SIZE)))
    expand = jnp.dot(idx32, e_ref[...],
                     preferred_element_type=jnp.float32)
    oh = (expand == lmod_ref[...]).astype(jnp.int8)
    acc = jnp.dot(oh, w_ref[...], preferred_element_type=jnp.int32)
    out_ref[...] = jnp.where(acc > 0, 1.0, -1.0).astype(jnp.float32)


def kernel(x, position_weight, level_weight):
    pos_p = jnp.concatenate(
        [position_weight, jnp.zeros((32 - _SIZE, _D), jnp.float32)], axis=0)
    lev_p = jnp.concatenate(
        [level_weight, jnp.zeros((_LP - _LEVELS, _D), jnp.float32)], axis=0)
    return pl.pallas_call(
        _body,
        grid=(_BATCH // _BB,),
        in_specs=[
            pl.BlockSpec((_BB, _SIZE), lambda i: (i, 0)),
            pl.BlockSpec((32, _D), lambda i: (0, 0)),
            pl.BlockSpec((_LP, _D), lambda i: (0, 0)),
        ],
        out_specs=pl.BlockSpec((_BB, _D), lambda i: (i, 0)),
        out_shape=jax.ShapeDtypeStruct((_BATCH, _D), jnp.float32),
        scratch_shapes=[pltpu.VMEM((_K, _D), jnp.int8),
                        pltpu.VMEM((32, _K), jnp.bfloat16),
                        pltpu.VMEM((_BB, _K), jnp.float32)],
    )(x, pos_p, lev_p)


# speedup vs baseline: 1.2785x; 1.2785x over previous
import jax
import jax.numpy as jnp
from jax.experimental import pallas as pl
from jax.experimental.pallas import tpu as pltpu

_SIZE = 26
_D = 2048
_LEVELS = 100
_LP = 104
_K = _SIZE * _LP
_BATCH = 1024
_BB = 256


def _body(x_ref, pos_ref, lev_ref, out_ref, w_ref, e_ref, lmod_ref):
    i = pl.program_id(0)

    @pl.when(i == 0)
    def _build_tables():
        lev = lev_ref[...]
        rows = jax.lax.broadcasted_iota(jnp.int32, (_LP, _D), 0)
        lev = jnp.where(rows < _LEVELS, lev, 0.0)
        for s in range(_SIZE):
            p = pos_ref[s:s + 1, :]
            w_ref[s * _LP:(s + 1) * _LP, :] = (lev * p).astype(jnp.bfloat16)
        js = jax.lax.broadcasted_iota(jnp.int32, (32, _K), 1) // _LP
        ss = jax.lax.broadcasted_iota(jnp.int32, (32, _K), 0)
        e_ref[...] = jnp.where(js == ss, 1.0, 0.0).astype(jnp.bfloat16)
        cols = jax.lax.broadcasted_iota(jnp.int32, (_BB, _K), 1)
        lmod_ref[...] = (cols % _LP).astype(jnp.float32)

    xb = x_ref[...]
    idx = jnp.clip(jnp.round(xb * (_LEVELS - 1)).astype(jnp.int32),
                   0, _LEVELS - 1)
    idx32 = jnp.pad(idx.astype(jnp.bfloat16), ((0, 0), (0, 32 - _SIZE)))
    expand = jnp.dot(idx32, e_ref[...],
                     preferred_element_type=jnp.float32)
    oh = (expand == lmod_ref[...]).astype(jnp.bfloat16)
    acc = jnp.dot(oh, w_ref[...], preferred_element_type=jnp.float32)
    out_ref[...] = jnp.where(acc > 0, 1.0, -1.0).astype(jnp.float32)


def kernel(x, position_weight, level_weight):
    return pl.pallas_call(
        _body,
        grid=(_BATCH // _BB,),
        in_specs=[
            pl.BlockSpec((_BB, _SIZE), lambda i: (i, 0)),
            pl.BlockSpec((32, _D), lambda i: (0, 0)),
            pl.BlockSpec((_LP, _D), lambda i: (0, 0)),
        ],
        out_specs=pl.BlockSpec((_BB, _D), lambda i: (i, 0)),
        out_shape=jax.ShapeDtypeStruct((_BATCH, _D), jnp.float32),
        scratch_shapes=[pltpu.VMEM((_K, _D), jnp.bfloat16),
                        pltpu.VMEM((32, _K), jnp.bfloat16),
                        pltpu.VMEM((_BB, _K), jnp.float32)],
    )(x, position_weight, level_weight)
